# Initial kernel scaffold; baseline (speedup 1.0000x reference)
import jax
import jax.numpy as jnp
from jax.experimental import pallas as pl


def kernel(pred, target):
    raise NotImplementedError("write your pallas kernel here")


# speedup vs baseline: 2.5459x; 2.5459x over previous
import jax
import jax.numpy as jnp
from jax import lax
from jax.experimental import pallas as pl
from jax.experimental.pallas import tpu as pltpu

_EPS = 1e-6


def _make_body(H, W):
    SENT = H * W
    BIG = 2**30

    def shift_up(x, fill):
        return jnp.concatenate(
            [x[1:, :], jnp.full((1, W), fill, x.dtype)], axis=0)

    def shift_down(x, fill):
        return jnp.concatenate(
            [jnp.full((1, W), fill, x.dtype), x[:-1, :]], axis=0)

    def shift_left(x, fill):
        return jnp.concatenate(
            [x[:, 1:], jnp.full((H, 1), fill, x.dtype)], axis=1)

    def shift_right(x, fill):
        return jnp.concatenate(
            [jnp.full((H, 1), fill, x.dtype), x[:, :-1]], axis=1)

    def body(p_ref, t_ref, out_ref):
        b = pl.program_id(0)
        nb_ = pl.num_programs(0)
        p = p_ref[0]
        t = t_ref[0]
        fg = (p + t) > 0.0

        rows = lax.broadcasted_iota(jnp.int32, (H, W), 0)
        cols = lax.broadcasted_iota(jnp.int32, (H, W), 1)
        labels0 = jnp.where(fg, rows * W + cols, SENT)

        def cc_cond(st):
            return st[3]

        def cc_body(st):
            labels, tch, it, _ = st
            it = it + 1
            nbmin = jnp.minimum(
                jnp.minimum(shift_up(labels, SENT), shift_down(labels, SENT)),
                jnp.minimum(shift_left(labels, SENT), shift_right(labels, SENT)))
            new = jnp.where(fg, jnp.minimum(labels, nbmin), SENT)
            ch = new != labels
            tch = jnp.where(ch, it, tch)
            return new, tch, it, jnp.any(ch)

        _, tch, _, _ = lax.while_loop(
            cc_cond, cc_body,
            (labels0, jnp.zeros((H, W), jnp.int32), jnp.int32(0),
             jnp.array(True)))

        is_root = fg & (tch == 0)
        d = jnp.where(fg, tch, BIG)

        dN = shift_down(d, BIG)
        dS = shift_up(d, BIG)
        dW = shift_right(d, BIG)
        dE = shift_left(d, BIG)
        tgt = d - 1
        sender = fg & jnp.logical_not(is_root)
        sN = sender & (dN == tgt)
        sW = sender & (dW == tgt) & jnp.logical_not(sN)
        sE = sender & (dE == tgt) & jnp.logical_not(sN | sW)
        sS = sender & (dS == tgt) & jnp.logical_not(sN | sW | sE)

        zero = jnp.float32(0.0)
        I0 = jnp.where(fg, p * t, zero)
        D0 = jnp.where(fg, p * t + 2.0 - p - t, zero)

        M = jnp.max(jnp.where(fg, tch, 0))

        def drain_step(v):
            inc = (shift_up(jnp.where(sN, v, zero), zero)
                   + shift_down(jnp.where(sS, v, zero), zero)
                   + shift_left(jnp.where(sW, v, zero), zero)
                   + shift_right(jnp.where(sE, v, zero), zero))
            return inc + jnp.where(is_root, v, zero)

        def drain_body(_, carry):
            iacc, dacc = carry
            return drain_step(iacc), drain_step(dacc)

        Iacc, Dacc = lax.fori_loop(0, M, drain_body, (I0, D0))

        rootf = jnp.where(is_root, jnp.float32(1.0), zero)
        num = jnp.sum(rootf)
        ratio = (Iacc + _EPS) / (Dacc + _EPS)
        tsum = jnp.sum(jnp.where(is_root, ratio, zero))
        loss_b = jnp.where(num == 0.0, jnp.float32(1.0),
                           1.0 - tsum / jnp.maximum(num, 1.0))

        @pl.when(b == 0)
        def _():
            out_ref[0, 0] = 0.0

        out_ref[0, 0] += loss_b / jnp.float32(nb_)

    return body


def kernel(pred, target):
    if pred.shape != target.shape:
        raise ValueError(
            f'Pred shape {pred.shape} must match target shape {target.shape}')
    B, H, W = pred.shape
    body = _make_body(H, W)
    out = pl.pallas_call(
        body,
        grid=(B,),
        in_specs=[
            pl.BlockSpec((1, H, W), lambda b: (b, 0, 0)),
            pl.BlockSpec((1, H, W), lambda b: (b, 0, 0)),
        ],
        out_specs=pl.BlockSpec(memory_space=pltpu.SMEM),
        out_shape=jax.ShapeDtypeStruct((1, 1), jnp.float32),
    )(pred, target)
    return out.reshape(())
